# baseline (speedup 1.0000x reference)
import functools

import jax
import jax.numpy as jnp
from jax import lax
from jax.experimental import pallas as pl
from jax.experimental.pallas import tpu as pltpu
from jax.experimental.pallas import tpu_sc as plsc

N = 10000
E = 320000
D_IN = 128
D_EMB = 128
D_OUT = 64

NC = 2
NS = 16
LANES = 16
NW = NC * NS
E_PER = E // NW
N_PAD = 10240
Z_PER = N_PAD // NS

BN = 2560
N_BLOCKS = N_PAD // BN


def _sc_histograms(edge_index, ew1, ew2):
    mesh = plsc.VectorSubcoreMesh(core_axis_name="c", subcore_axis_name="s")

    @functools.partial(
        pl.kernel,
        out_type=(
            jax.ShapeDtypeStruct((NC, N_PAD), jnp.float32),
            jax.ShapeDtypeStruct((NC, N_PAD), jnp.float32),
        ),
        mesh=mesh,
        scratch_types=[
            pltpu.VMEM((E_PER,), jnp.int32),
            pltpu.VMEM((E_PER,), jnp.float32),
            pltpu.VMEM((E_PER,), jnp.float32),
            pltpu.VMEM((Z_PER,), jnp.float32),
            pltpu.VMEM_SHARED((N_PAD,), jnp.float32),
            pltpu.VMEM_SHARED((N_PAD,), jnp.float32),
            pltpu.SemaphoreType.DMA,
            pltpu.SemaphoreType.DMA,
            pltpu.SemaphoreType.DMA,
            pltpu.SemaphoreType.DMA,
            pltpu.SemaphoreType.DMA,
        ],
    )
    def hist_kernel(idx_hbm, ew1_hbm, ew2_hbm, out1_hbm, out2_hbm,
                    idx_v, w1_v, w2_v, z_v, h1_s, h2_s,
                    sem1, sem2, sem3, sem4, sem5):
        cid = lax.axis_index("c")
        sid = lax.axis_index("s")
        base = (cid * NS + sid) * E_PER
        cp1 = pltpu.async_copy(idx_hbm.at[pl.ds(base, E_PER)], idx_v, sem1)
        cp2 = pltpu.async_copy(ew1_hbm.at[pl.ds(base, E_PER)], w1_v, sem2)
        cp3 = pltpu.async_copy(ew2_hbm.at[pl.ds(base, E_PER)], w2_v, sem3)

        zero = jnp.zeros((LANES,), jnp.float32)

        @pl.loop(0, Z_PER, step=LANES)
        def _(i):
            z_v[pl.ds(i, LANES)] = zero

        slc = pl.ds(sid * Z_PER, Z_PER)
        pltpu.sync_copy(z_v, h1_s.at[slc])
        pltpu.sync_copy(z_v, h2_s.at[slc])
        plsc.subcore_barrier()

        cp1.wait()
        cp2.wait()
        cp3.wait()
        sc1 = pltpu.async_copy(w1_v, h1_s.at[idx_v], sem4, add=True)
        sc2 = pltpu.async_copy(w2_v, h2_s.at[idx_v], sem5, add=True)
        sc1.wait()
        sc2.wait()
        plsc.subcore_barrier()

        pltpu.sync_copy(h1_s.at[slc], out1_hbm.at[cid, slc])
        pltpu.sync_copy(h2_s.at[slc], out2_hbm.at[cid, slc])

    return hist_kernel(edge_index, ew1, ew2)


def _mm1_body(ft_ref, ws1_ref, t_ref):
    dot = functools.partial(jnp.dot, preferred_element_type=jnp.float32)
    t_ref[...] = dot(ws1_ref[...], ft_ref[...]).astype(jnp.bfloat16)


def _mm1(ft, ws1):
    return pl.pallas_call(
        _mm1_body,
        grid=(N_BLOCKS,),
        in_specs=[
            pl.BlockSpec((D_IN, BN), lambda i: (0, i)),
            pl.BlockSpec((3 * D_EMB, D_IN), lambda i: (0, 0)),
        ],
        out_specs=pl.BlockSpec((3 * D_EMB, BN), lambda i: (0, i)),
        out_shape=jax.ShapeDtypeStruct((3 * D_EMB, N_PAD), jnp.bfloat16),
    )(ft, ws1)


def _mm2_body(t_ref, w1_ref, w2_ref, ws2_ref, bias1_ref, bias2_ref, o_ref):
    t = t_ref[...]
    w1 = (w1_ref[0, :] + w1_ref[1, :])[None, :]
    w2 = (w2_ref[0, :] + w2_ref[1, :])[None, :]
    dot = functools.partial(jnp.dot, preferred_element_type=jnp.float32)
    x = (t[:D_EMB, :].astype(jnp.float32)
         + t[D_EMB:2 * D_EMB, :].astype(jnp.float32) * w1
         + t[2 * D_EMB:, :].astype(jnp.float32) * w2
         + bias1_ref[...])
    u = dot(ws2_ref[...], x.astype(jnp.bfloat16))
    z = (u[:D_OUT, :]
         + u[D_OUT:2 * D_OUT, :] * w1
         + u[2 * D_OUT:, :] * w2
         + bias2_ref[...])
    o_ref[...] = z


def _mm2(t, w1p, w2p, ws2, bias1, bias2):
    full = lambda r, c: pl.BlockSpec((r, c), lambda i: (0, 0))
    return pl.pallas_call(
        _mm2_body,
        grid=(N_BLOCKS,),
        in_specs=[
            pl.BlockSpec((3 * D_EMB, BN), lambda i: (0, i)),
            pl.BlockSpec((NC, BN), lambda i: (0, i)),
            pl.BlockSpec((NC, BN), lambda i: (0, i)),
            full(3 * D_OUT, D_EMB),
            full(D_EMB, 1),
            full(D_OUT, 1),
        ],
        out_specs=pl.BlockSpec((D_OUT, BN), lambda i: (0, i)),
        out_shape=jax.ShapeDtypeStruct((D_OUT, N), jnp.float32),
    )(t, w1p, w2p, ws2, bias1, bias2)


def kernel(features, edge_index1, edge_index2, edge_weight1, edge_weight2,
           ib1_ln_W, ib1_ln_b, ib1_c1_W, ib1_c1_b, ib1_c2_W, ib1_c2_b,
           ib2_ln_W, ib2_ln_b, ib2_c1_W, ib2_c1_b, ib2_c2_W, ib2_c2_b):
    w1p, w2p = _sc_histograms(edge_index1, edge_weight1, edge_weight2)
    bf = jnp.bfloat16
    ft = features.T.astype(bf)
    ws1 = jnp.concatenate([ib1_ln_W, ib1_c1_W.T, ib1_c2_W.T], axis=0).astype(bf)
    ws2 = jnp.concatenate([ib2_ln_W, ib2_c1_W.T, ib2_c2_W.T], axis=0).astype(bf)
    bias1 = (ib1_ln_b + ib1_c1_b + ib1_c2_b).reshape(D_EMB, 1)
    bias2 = (ib2_ln_b + ib2_c1_b + ib2_c2_b).reshape(D_OUT, 1)
    t = _mm1(ft, ws1)
    zt = _mm2(t, w1p, w2p, ws2, bias1, bias2)
    return zt.T


# speedup vs baseline: 1.1034x; 1.1034x over previous
import functools

import jax
import jax.numpy as jnp
from jax import lax
from jax.experimental import pallas as pl
from jax.experimental.pallas import tpu as pltpu
from jax.experimental.pallas import tpu_sc as plsc

N = 10000
E = 320000
D_IN = 128
D_EMB = 128
D_OUT = 64

NC = 2
NS = 16
LANES = 16
NW = NC * NS
E_PER = E // NW
N_PAD = 10240
Z_PER = N_PAD // NS

BN = 2560
N_BLOCKS = N_PAD // BN


def _sc_histograms(edge_index, ew1, ew2):
    mesh = plsc.VectorSubcoreMesh(core_axis_name="c", subcore_axis_name="s")

    @functools.partial(
        pl.kernel,
        out_type=(
            jax.ShapeDtypeStruct((NC, N_PAD), jnp.float32),
            jax.ShapeDtypeStruct((NC, N_PAD), jnp.float32),
        ),
        mesh=mesh,
        scratch_types=[
            pltpu.VMEM((E_PER,), jnp.int32),
            pltpu.VMEM((E_PER,), jnp.float32),
            pltpu.VMEM((E_PER,), jnp.float32),
            pltpu.VMEM((Z_PER,), jnp.float32),
            pltpu.VMEM_SHARED((N_PAD,), jnp.float32),
            pltpu.VMEM_SHARED((N_PAD,), jnp.float32),
            pltpu.SemaphoreType.DMA,
            pltpu.SemaphoreType.DMA,
            pltpu.SemaphoreType.DMA,
            pltpu.SemaphoreType.DMA,
            pltpu.SemaphoreType.DMA,
        ],
    )
    def hist_kernel(idx_hbm, ew1_hbm, ew2_hbm, out1_hbm, out2_hbm,
                    idx_v, w1_v, w2_v, z_v, h1_s, h2_s,
                    sem1, sem2, sem3, sem4, sem5):
        cid = lax.axis_index("c")
        sid = lax.axis_index("s")
        base = (cid * NS + sid) * E_PER
        cp1 = pltpu.async_copy(idx_hbm.at[pl.ds(base, E_PER)], idx_v, sem1)
        cp2 = pltpu.async_copy(ew1_hbm.at[pl.ds(base, E_PER)], w1_v, sem2)
        cp3 = pltpu.async_copy(ew2_hbm.at[pl.ds(base, E_PER)], w2_v, sem3)

        zero = jnp.zeros((LANES,), jnp.float32)

        @pl.loop(0, Z_PER, step=LANES)
        def _(i):
            z_v[pl.ds(i, LANES)] = zero

        slc = pl.ds(sid * Z_PER, Z_PER)
        pltpu.sync_copy(z_v, h1_s.at[slc])
        pltpu.sync_copy(z_v, h2_s.at[slc])
        plsc.subcore_barrier()

        cp1.wait()
        cp2.wait()
        cp3.wait()
        sc1 = pltpu.async_copy(w1_v, h1_s.at[idx_v], sem4, add=True)
        sc2 = pltpu.async_copy(w2_v, h2_s.at[idx_v], sem5, add=True)
        sc1.wait()
        sc2.wait()
        plsc.subcore_barrier()

        pltpu.sync_copy(h1_s.at[slc], out1_hbm.at[cid, slc])
        pltpu.sync_copy(h2_s.at[slc], out2_hbm.at[cid, slc])

    return hist_kernel(edge_index, ew1, ew2)


def _dense_body(ft_ref, w1_ref, w2_ref, a1_ref, b1_ref, c1_ref,
                a2_ref, b2_ref, c2_ref, bias1_ref, bias2_ref, o_ref):
    f = ft_ref[...]
    w1 = (w1_ref[0, :] + w1_ref[1, :])[None, :]
    w2 = (w2_ref[0, :] + w2_ref[1, :])[None, :]
    dot = functools.partial(jnp.dot, preferred_element_type=jnp.float32)
    x = (dot(a1_ref[...], f)
         + dot(b1_ref[...], f) * w1
         + dot(c1_ref[...], f) * w2
         + bias1_ref[...])
    xb = x.astype(jnp.bfloat16)
    z = (dot(a2_ref[...], xb)
         + dot(b2_ref[...], xb) * w1
         + dot(c2_ref[...], xb) * w2
         + bias2_ref[...])
    o_ref[...] = z


def _dense(ft, w1p, w2p, a1, b1m, c1m, a2, b2m, c2m, bias1, bias2):
    full = lambda r, c: pl.BlockSpec((r, c), lambda i: (0, 0))
    return pl.pallas_call(
        _dense_body,
        grid=(N_BLOCKS,),
        in_specs=[
            pl.BlockSpec((D_IN, BN), lambda i: (0, i)),
            pl.BlockSpec((NC, BN), lambda i: (0, i)),
            pl.BlockSpec((NC, BN), lambda i: (0, i)),
            full(D_EMB, D_IN),
            full(D_EMB, D_IN),
            full(D_EMB, D_IN),
            full(D_OUT, D_EMB),
            full(D_OUT, D_EMB),
            full(D_OUT, D_EMB),
            full(D_EMB, 1),
            full(D_OUT, 1),
        ],
        out_specs=pl.BlockSpec((D_OUT, BN), lambda i: (0, i)),
        out_shape=jax.ShapeDtypeStruct((D_OUT, N), jnp.float32),
    )(ft, w1p, w2p, a1, b1m, c1m, a2, b2m, c2m, bias1, bias2)


def kernel(features, edge_index1, edge_index2, edge_weight1, edge_weight2,
           ib1_ln_W, ib1_ln_b, ib1_c1_W, ib1_c1_b, ib1_c2_W, ib1_c2_b,
           ib2_ln_W, ib2_ln_b, ib2_c1_W, ib2_c1_b, ib2_c2_W, ib2_c2_b):
    w1p, w2p = _sc_histograms(edge_index1, edge_weight1, edge_weight2)
    bf = jnp.bfloat16
    ft = features.T.astype(bf)
    bias1 = (ib1_ln_b + ib1_c1_b + ib1_c2_b).reshape(D_EMB, 1)
    bias2 = (ib2_ln_b + ib2_c1_b + ib2_c2_b).reshape(D_OUT, 1)
    zt = _dense(ft, w1p, w2p, ib1_ln_W.astype(bf), ib1_c1_W.T.astype(bf),
                ib1_c2_W.T.astype(bf), ib2_ln_W.astype(bf),
                ib2_c1_W.T.astype(bf), ib2_c2_W.T.astype(bf), bias1, bias2)
    return zt.T


# speedup vs baseline: 1.1258x; 1.0203x over previous
import functools

import jax
import jax.numpy as jnp
from jax import lax
from jax.experimental import pallas as pl
from jax.experimental.pallas import tpu as pltpu
from jax.experimental.pallas import tpu_sc as plsc

N = 10000
E = 320000
D_IN = 128
D_EMB = 128
D_OUT = 64

NC = 2
NS = 16
LANES = 16
NW = NC * NS
E_PER = E // NW
N_PAD = 10240
Z_PER = N_PAD // NS

BN = 5120
N_BLOCKS = N_PAD // BN


def _sc_histograms(edge_index, ew1, ew2):
    mesh = plsc.VectorSubcoreMesh(core_axis_name="c", subcore_axis_name="s")

    @functools.partial(
        pl.kernel,
        out_type=(
            jax.ShapeDtypeStruct((NC, N_PAD), jnp.float32),
            jax.ShapeDtypeStruct((NC, N_PAD), jnp.float32),
        ),
        mesh=mesh,
        scratch_types=[
            pltpu.VMEM((E_PER,), jnp.int32),
            pltpu.VMEM((E_PER,), jnp.float32),
            pltpu.VMEM((E_PER,), jnp.float32),
            pltpu.VMEM((Z_PER,), jnp.float32),
            pltpu.VMEM_SHARED((N_PAD,), jnp.float32),
            pltpu.VMEM_SHARED((N_PAD,), jnp.float32),
            pltpu.SemaphoreType.DMA,
            pltpu.SemaphoreType.DMA,
            pltpu.SemaphoreType.DMA,
            pltpu.SemaphoreType.DMA,
            pltpu.SemaphoreType.DMA,
        ],
    )
    def hist_kernel(idx_hbm, ew1_hbm, ew2_hbm, out1_hbm, out2_hbm,
                    idx_v, w1_v, w2_v, z_v, h1_s, h2_s,
                    sem1, sem2, sem3, sem4, sem5):
        cid = lax.axis_index("c")
        sid = lax.axis_index("s")
        base = (cid * NS + sid) * E_PER
        cp1 = pltpu.async_copy(idx_hbm.at[pl.ds(base, E_PER)], idx_v, sem1)
        cp2 = pltpu.async_copy(ew1_hbm.at[pl.ds(base, E_PER)], w1_v, sem2)
        cp3 = pltpu.async_copy(ew2_hbm.at[pl.ds(base, E_PER)], w2_v, sem3)

        zero = jnp.zeros((LANES,), jnp.float32)

        @pl.loop(0, Z_PER, step=LANES)
        def _(i):
            z_v[pl.ds(i, LANES)] = zero

        slc = pl.ds(sid * Z_PER, Z_PER)
        pltpu.sync_copy(z_v, h1_s.at[slc])
        pltpu.sync_copy(z_v, h2_s.at[slc])
        plsc.subcore_barrier()

        cp1.wait()
        cp2.wait()
        cp3.wait()
        sc1 = pltpu.async_copy(w1_v, h1_s.at[idx_v], sem4, add=True)
        sc2 = pltpu.async_copy(w2_v, h2_s.at[idx_v], sem5, add=True)
        sc1.wait()
        sc2.wait()
        plsc.subcore_barrier()

        pltpu.sync_copy(h1_s.at[slc], out1_hbm.at[cid, slc])
        pltpu.sync_copy(h2_s.at[slc], out2_hbm.at[cid, slc])

    return hist_kernel(edge_index, ew1, ew2)


def _dense_body(ft_ref, w1_ref, w2_ref, a1_ref, b1_ref, c1_ref,
                a2_ref, b2_ref, c2_ref, bias1_ref, bias2_ref, o_ref):
    f = ft_ref[...]
    w1 = (w1_ref[0, :] + w1_ref[1, :])[None, :]
    w2 = (w2_ref[0, :] + w2_ref[1, :])[None, :]
    dot = functools.partial(jnp.dot, preferred_element_type=jnp.float32)
    x = (dot(a1_ref[...], f)
         + dot(b1_ref[...], f) * w1
         + dot(c1_ref[...], f) * w2
         + bias1_ref[...])
    xb = x.astype(jnp.bfloat16)
    z = (dot(a2_ref[...], xb)
         + dot(b2_ref[...], xb) * w1
         + dot(c2_ref[...], xb) * w2
         + bias2_ref[...])
    o_ref[...] = z


def _dense(ft, w1p, w2p, a1, b1m, c1m, a2, b2m, c2m, bias1, bias2):
    full = lambda r, c: pl.BlockSpec((r, c), lambda i: (0, 0))
    return pl.pallas_call(
        _dense_body,
        grid=(N_BLOCKS,),
        in_specs=[
            pl.BlockSpec((D_IN, BN), lambda i: (0, i)),
            pl.BlockSpec((NC, BN), lambda i: (0, i)),
            pl.BlockSpec((NC, BN), lambda i: (0, i)),
            full(D_EMB, D_IN),
            full(D_EMB, D_IN),
            full(D_EMB, D_IN),
            full(D_OUT, D_EMB),
            full(D_OUT, D_EMB),
            full(D_OUT, D_EMB),
            full(D_EMB, 1),
            full(D_OUT, 1),
        ],
        out_specs=pl.BlockSpec((D_OUT, BN), lambda i: (0, i)),
        out_shape=jax.ShapeDtypeStruct((D_OUT, N), jnp.float32),
    )(ft, w1p, w2p, a1, b1m, c1m, a2, b2m, c2m, bias1, bias2)


def kernel(features, edge_index1, edge_index2, edge_weight1, edge_weight2,
           ib1_ln_W, ib1_ln_b, ib1_c1_W, ib1_c1_b, ib1_c2_W, ib1_c2_b,
           ib2_ln_W, ib2_ln_b, ib2_c1_W, ib2_c1_b, ib2_c2_W, ib2_c2_b):
    w1p, w2p = _sc_histograms(edge_index1, edge_weight1, edge_weight2)
    bf = jnp.bfloat16
    ft = features.T.astype(bf)
    bias1 = (ib1_ln_b + ib1_c1_b + ib1_c2_b).reshape(D_EMB, 1)
    bias2 = (ib2_ln_b + ib2_c1_b + ib2_c2_b).reshape(D_OUT, 1)
    zt = _dense(ft, w1p, w2p, ib1_ln_W.astype(bf), ib1_c1_W.T.astype(bf),
                ib1_c2_W.T.astype(bf), ib2_ln_W.astype(bf),
                ib2_c1_W.T.astype(bf), ib2_c2_W.T.astype(bf), bias1, bias2)
    return zt.T
